# Initial kernel scaffold; baseline (speedup 1.0000x reference)
import jax
import jax.numpy as jnp
from jax.experimental import pallas as pl


def kernel(logits_weight, edge_ids):
    raise NotImplementedError("write your pallas kernel here")


# speedup vs baseline: 18.7487x; 18.7487x over previous
import functools

import jax
import jax.numpy as jnp
from jax import lax
from jax.experimental import pallas as pl
from jax.experimental.pallas import tpu as pltpu
from jax.experimental.pallas import tpu_sc as plsc

N = 6_400_000
K_KEEP = N // 2
NW = 32
PER_W = N // NW
BLK = 8_000
NBLK = PER_W // BLK
VPB = BLK // 16

NB0 = 2048
NB1 = 2048
NB2 = 1024

_MESH = plsc.VectorSubcoreMesh(core_axis_name="c", subcore_axis_name="s")
_SC_PARAMS = pltpu.CompilerParams(needs_layout_passes=False)


def _wid():
    return lax.axis_index("s") * 2 + lax.axis_index("c")


def _lane_reduce(hist_v, red_v, nbins, lane):
    def group_body(g, carry):
        acc = jnp.zeros((16,), jnp.int32)
        for j in range(16):
            s = jnp.sum(hist_v[pl.ds((g * 16 + j) * 16, 16)])
            acc = jnp.where(lane == j, jnp.full((16,), s, jnp.int32), acc)
        red_v[pl.ds(g * 16, 16)] = acc
        return carry

    lax.fori_loop(0, nbins // 16, group_body, 0)


@functools.partial(
    pl.kernel,
    out_type=jax.ShapeDtypeStruct((N,), jnp.float32),
    mesh=_MESH,
    compiler_params=_SC_PARAMS,
    scratch_types=[
        pltpu.VMEM((BLK,), jnp.int32),
        pltpu.VMEM((BLK,), jnp.float32),
        pltpu.SemaphoreType.DMA,
    ],
)
def _k1_gather(table, ids, probs_out, idx_v, val_v, sem):
    wid = _wid()
    base = wid * PER_W

    def blk_body(b, carry):
        off = base + b * BLK
        pltpu.sync_copy(ids.at[pl.ds(off, BLK)], idx_v)
        pltpu.async_copy(table.at[idx_v], val_v, sem).wait()

        def vec_body(j, c2):
            x = val_v[pl.ds(j * 16, 16)]
            val_v[pl.ds(j * 16, 16)] = 1.0 / (1.0 + jnp.exp(-x))
            return c2

        lax.fori_loop(0, VPB, vec_body, 0)
        pltpu.sync_copy(val_v, probs_out.at[pl.ds(off, BLK)])
        return carry

    lax.fori_loop(0, NBLK, blk_body, 0)


def _make_hist_level(nbins, shift, pshift):
    @functools.partial(
        pl.kernel,
        out_type=jax.ShapeDtypeStruct((NW, nbins), jnp.int32),
        mesh=_MESH,
    compiler_params=_SC_PARAMS,
        scratch_types=[
            pltpu.VMEM((BLK,), jnp.int32),
            pltpu.VMEM((nbins * 16,), jnp.int32),
            pltpu.VMEM((nbins,), jnp.int32),
            pltpu.VMEM((2, 16), jnp.int32),
        ],
    )
    def _hist_level(keys, zeros, params, hist_out, val_v, hist_v, red_v, par_v):
        wid = _wid()
        base = wid * PER_W
        lane = lax.iota(jnp.int32, 16)
        ones = jnp.ones((16,), jnp.int32)
        pltpu.sync_copy(zeros.at[pl.ds(0, nbins * 16)], hist_v)
        pltpu.sync_copy(params, par_v)
        want = jnp.right_shift(par_v[0, :], pshift)

        def blk_body(b, carry):
            off = base + b * BLK
            pltpu.sync_copy(keys.at[pl.ds(off, BLK)], val_v)

            def vec_body(j, c2):
                key = val_v[pl.ds(j * 16, 16)]
                match = jnp.right_shift(key, pshift) == want
                bins = jnp.right_shift(key, shift) & (nbins - 1)
                plsc.addupdate_scatter(hist_v, [bins * 16 + lane], ones,
                                       mask=match)
                return c2

            lax.fori_loop(0, VPB, vec_body, 0)
            return carry

        lax.fori_loop(0, NBLK, blk_body, 0)
        _lane_reduce(hist_v, red_v, nbins, lane)
        pltpu.sync_copy(red_v, hist_out.at[wid])

    return _hist_level


_h0_hist = _make_hist_level(NB0, 21, 31)
_h1_hist = _make_hist_level(NB1, 10, 21)
_h2_hist = _make_hist_level(NB2, 0, 10)


@functools.partial(
    pl.kernel,
    out_type=[
        jax.ShapeDtypeStruct((N,), jnp.float32),
        jax.ShapeDtypeStruct((N,), jnp.float32),
    ],
    mesh=_MESH,
    compiler_params=_SC_PARAMS,
    scratch_types=[
        pltpu.VMEM((BLK,), jnp.float32),
        pltpu.VMEM((BLK,), jnp.float32),
        pltpu.VMEM((BLK,), jnp.float32),
        pltpu.VMEM((1, 16), jnp.float32),
        pltpu.VMEM((NW, 16), jnp.int32),
    ],
)
def _k4_mask(probs, tpar, bpar, hard_out, soft_out,
             val_v, hard_v, soft_v, tpar_v, bpar_v):
    wid = _wid()
    base = wid * PER_W
    pltpu.sync_copy(tpar, tpar_v)
    pltpu.sync_copy(bpar, bpar_v)
    t_v = tpar_v[0, :]
    b_v = jnp.zeros((16,), jnp.int32)
    for i in range(NW):
        b_v = jnp.where(wid == i, bpar_v[i, :], b_v)

    def blk_body(b, seen):
        off = base + b * BLK
        pltpu.sync_copy(probs.at[pl.ds(off, BLK)], val_v)

        def vec_body(j, seen2):
            p = val_v[pl.ds(j * 16, 16)]
            gt = p > t_v
            tie = p == t_v
            t_i = tie.astype(jnp.int32)
            excl = plsc.cumsum(t_i) - t_i
            rank = excl + jnp.full((16,), seen2, jnp.int32)
            keep = jnp.logical_or(gt, jnp.logical_and(tie, rank < b_v))
            hard = jnp.where(keep, jnp.float32(1.0), jnp.float32(0.0))
            hard_v[pl.ds(j * 16, 16)] = hard
            soft_v[pl.ds(j * 16, 16)] = (hard - p) + p
            return seen2 + jnp.sum(t_i)

        seen = lax.fori_loop(0, VPB, vec_body, seen)
        pltpu.sync_copy(hard_v, hard_out.at[pl.ds(off, BLK)])
        pltpu.sync_copy(soft_v, soft_out.at[pl.ds(off, BLK)])
        return seen

    lax.fori_loop(0, NBLK, blk_body, jnp.int32(0))


def _suffix_sum(x, nbins):
    sh = 1
    while sh < nbins:
        pad = jnp.zeros((1, sh), jnp.int32)
        x = x + jnp.concatenate([x[:, sh:], pad], axis=1)
        sh *= 2
    return x


def _make_select_tc(nbins, shift):
    def body(hist_ref, params_ref, out_ref):
        prefix = params_ref[0, 0]
        k_rem = params_ref[1, 0]
        h = jnp.sum(hist_ref[...], axis=0, keepdims=True)
        s = _suffix_sum(h, nbins)
        iota = lax.broadcasted_iota(jnp.int32, (1, nbins), 1)
        mask = s >= k_rem
        hi = jnp.max(jnp.where(mask, iota, -1))
        s_h = jnp.min(jnp.where(mask, s, jnp.int32(2**31 - 1)))
        h_h = jnp.sum(jnp.where(iota == hi, h, 0))
        count_gt = s_h - h_h
        new_prefix = prefix | lax.shift_left(hi, shift)
        new_k = k_rem - count_gt
        out_ref[...] = jnp.concatenate(
            [jnp.full((1, 16), new_prefix, jnp.int32),
             jnp.full((1, 16), new_k, jnp.int32)], axis=0)

    return pl.pallas_call(
        body, out_shape=jax.ShapeDtypeStruct((2, 16), jnp.int32))


_t1_select = _make_select_tc(NB0, 21)
_t2_select = _make_select_tc(NB1, 10)


def _t3_budget_body(hist_ref, params_ref, tout_ref, bout_ref):
    prefix = params_ref[0, 0]
    k_rem = params_ref[1, 0]
    hh = hist_ref[...]
    h = jnp.sum(hh, axis=0, keepdims=True)
    s = _suffix_sum(h, NB2)
    iota = lax.broadcasted_iota(jnp.int32, (1, NB2), 1)
    mask = s >= k_rem
    lo = jnp.max(jnp.where(mask, iota, -1))
    s_h = jnp.min(jnp.where(mask, s, jnp.int32(2**31 - 1)))
    h_h = jnp.sum(jnp.where(iota == lo, h, 0))
    r = k_rem - (s_h - h_h)
    t_bits = prefix | lo
    c = jnp.sum(jnp.where(iota == lo, hh, 0), axis=1, keepdims=True)
    x = c
    sh = 1
    while sh < NW:
        pad = jnp.zeros((sh, 1), jnp.int32)
        x = x + jnp.concatenate([pad, x[:-sh, :]], axis=0)
        sh *= 2
    p_excl = x - c
    budget = jnp.clip(r - p_excl, 0, c)
    tout_ref[...] = jnp.full((1, 16), t_bits, jnp.int32)
    bout_ref[...] = jnp.broadcast_to(budget, (NW, 16))


_t3_budget = pl.pallas_call(
    _t3_budget_body,
    out_shape=[jax.ShapeDtypeStruct((1, 16), jnp.int32),
               jax.ShapeDtypeStruct((NW, 16), jnp.int32)])


def kernel(logits_weight, edge_ids):
    table = logits_weight.reshape(-1)
    zeros_hist = jnp.zeros((NB0 * 16,), jnp.int32)
    probs = _k1_gather(table, edge_ids)
    keys = lax.bitcast_convert_type(probs, jnp.int32)
    params0 = jnp.concatenate(
        [jnp.zeros((1, 16), jnp.int32),
         jnp.full((1, 16), K_KEEP, jnp.int32)], axis=0)
    hist0 = _h0_hist(keys, zeros_hist, params0)
    p1 = _t1_select(hist0, params0)
    hist1 = _h1_hist(keys, zeros_hist, p1)
    p2 = _t2_select(hist1, p1)
    hist2 = _h2_hist(keys, zeros_hist, p2)
    t_bits, budgets = _t3_budget(hist2, p2)
    t_f32 = lax.bitcast_convert_type(t_bits, jnp.float32)
    hard, soft = _k4_mask(probs, t_f32, budgets)
    return probs, soft, hard
